# baseline (speedup 1.0000x reference)
import functools

import jax
import jax.numpy as jnp
from jax import lax
from jax.experimental import pallas as pl
from jax.experimental.pallas import tpu as pltpu
from jax.experimental.pallas import tpu_sc as plsc

BB, LL, EE = 16, 4096, 1024
VV = 32000
NWW = 2048
BM = 256


NG = 4
GR = BB // NG


def _make_gather():
    info = plsc.get_sparse_core_info()
    nc, ns = info.num_cores, info.num_subcores
    nworkers = nc * ns
    per_tile = GR * LL // nworkers
    tbatch = 56
    nfull = per_tile // tbatch
    tail = per_tile - nfull * tbatch

    mesh = plsc.VectorSubcoreMesh(core_axis_name="c", subcore_axis_name="s")

    @functools.partial(
        pl.kernel,
        mesh=mesh,
        out_type=jax.ShapeDtypeStruct((GR * LL, EE), jnp.float32),
        scratch_types=[
            pltpu.VMEM((tbatch,), jnp.int32),
            pltpu.VMEM((tbatch,), jnp.int32),
            pltpu.VMEM((tbatch, EE), jnp.float32),
            pltpu.VMEM((tbatch, EE), jnp.float32),
            pltpu.SemaphoreType.DMA,
            pltpu.SemaphoreType.DMA,
            pltpu.SemaphoreType.DMA,
            pltpu.SemaphoreType.DMA,
        ],
    )
    def gather_k(table, idsflat, out, idx0, idx1, rows0, rows1,
                 gsem0, gsem1, wsem0, wsem1):
        c = lax.axis_index("c")
        s = lax.axis_index("s")
        base = (s * nc + c) * per_tile
        idx = (idx0, idx1)
        rows = (rows0, rows1)
        gsem = (gsem0, gsem1)
        wsem = (wsem0, wsem1)

        def pair_body(g, carry):
            for u in (0, 1):
                i = g * 2 + u

                @pl.when(jnp.logical_and(i >= 2, i < nfull))
                def _():
                    pltpu.make_async_copy(
                        rows[u],
                        out.at[pl.ds(base + (i - 2) * tbatch, tbatch)],
                        wsem[u]).wait()

                @pl.when(i < nfull)
                def _():
                    pltpu.sync_copy(
                        idsflat.at[pl.ds(base + i * tbatch, tbatch)], idx[u])
                    pltpu.async_copy(table.at[idx[u]], rows[u], gsem[u])

                @pl.when(jnp.logical_and(i >= 1, i <= nfull))
                def _():
                    v = 1 - u
                    pltpu.make_async_copy(table.at[idx[v]], rows[v],
                                          gsem[v]).wait()
                    pltpu.async_copy(
                        rows[v],
                        out.at[pl.ds(base + (i - 1) * tbatch, tbatch)],
                        wsem[v])
            return carry

        lax.fori_loop(0, (nfull + 2 + 1) // 2, pair_body, 0)
        pltpu.make_async_copy(
            rows0, out.at[pl.ds(base, tbatch)], wsem0).wait()
        pltpu.make_async_copy(
            rows1, out.at[pl.ds(base, tbatch)], wsem1).wait()
        t0 = base + nfull * tbatch
        pltpu.sync_copy(idsflat.at[pl.ds(t0, tail)], idx0.at[pl.ds(0, tail)])
        pltpu.async_copy(table.at[idx0.at[pl.ds(0, tail)]],
                         rows0.at[pl.ds(0, tail)], gsem0).wait()
        pltpu.sync_copy(rows0.at[pl.ds(0, tail)], out.at[pl.ds(t0, tail)])

    return gather_k


CK = 512


def make_tc_body(bm, ll, ee, ck=CK, has_prev=False):
    def tc_body(raw_ref, seg_ref, w_ref, b_ref, pad_ref, *rest):
        if has_prev:
            _prev_ref, out_ref, acc_ref, cnt_ref = rest
        else:
            out_ref, acc_ref, cnt_ref = rest
        i = pl.program_id(1)
        w0 = i * bm
        wids = lax.broadcasted_iota(jnp.int32, (bm, 1), 0) + w0
        acc_ref[...] = jnp.zeros((bm, ee), jnp.float32)
        cnt_ref[...] = jnp.zeros((bm, 128), jnp.float32)
        for k in range(ll // ck):
            lo_k = seg_ref[0, 0, k * ck]
            hi_k = seg_ref[0, 0, (k + 1) * ck - 1]

            @pl.when(jnp.logical_and(hi_k >= w0, lo_k < w0 + bm))
            def _():
                segc = seg_ref[0, :, k * ck:(k + 1) * ck]
                eqk = segc == wids
                ohk = eqk.astype(jnp.bfloat16)
                rawk = raw_ref[0, k * ck:(k + 1) * ck, :].astype(jnp.bfloat16)
                acc_ref[...] += jnp.dot(ohk, rawk,
                                        preferred_element_type=jnp.float32)
                cntk = jnp.sum(eqk.astype(jnp.float32), axis=1, keepdims=True)
                cnt_ref[...] += jnp.broadcast_to(cntk, (bm, 128))

        cnt = cnt_ref[:, 0:1]
        inv = 1.0 / jnp.maximum(cnt, 1.0)
        y = jnp.dot((acc_ref[...] * inv).astype(jnp.bfloat16), w_ref[...],
                    preferred_element_type=jnp.float32) + b_ref[...]
        out_ref[0] = jnp.where(cnt > 0.0, y, pad_ref[...])
    return tc_body


def _pool_project_group(raw_g, seg_g, w_bf, b2, pad2, g0, prev):
    gr, ll, ee = raw_g.shape
    nw = NWW
    in_specs = [
        pl.BlockSpec((1, ll, ee), lambda b, i: (b, 0, 0)),
        pl.BlockSpec((1, 1, ll), lambda b, i: (b, 0, 0)),
        pl.BlockSpec((ee, ee), lambda b, i: (0, 0)),
        pl.BlockSpec((1, ee), lambda b, i: (0, 0)),
        pl.BlockSpec((1, ee), lambda b, i: (0, 0)),
    ]
    args = [raw_g, seg_g.reshape(gr, 1, ll), w_bf, b2, pad2]
    aliases = {}
    if prev is not None:
        in_specs.append(pl.BlockSpec(memory_space=pl.ANY))
        args.append(prev)
        aliases = {5: 0}
    return pl.pallas_call(
        make_tc_body(BM, ll, ee, has_prev=prev is not None),
        grid=(gr, nw // BM),
        in_specs=in_specs,
        out_specs=pl.BlockSpec((1, BM, ee),
                               lambda b, i: (b + g0 * GR, i, 0)),
        out_shape=jax.ShapeDtypeStruct((BB, nw, ee), jnp.float32),
        input_output_aliases=aliases,
        scratch_shapes=[
            pltpu.VMEM((BM, ee), jnp.float32),
            pltpu.VMEM((BM, 128), jnp.float32),
        ],
    )(*args)


_gather_fn = None


def kernel(subword_ids, segment_ids, W_embed, W_proj, b_proj, padding_vec):
    global _gather_fn
    if _gather_fn is None:
        _gather_fn = _make_gather()
    ids32 = subword_ids.astype(jnp.int32)
    w_bf = W_proj.astype(jnp.bfloat16)
    b2 = b_proj.reshape(1, EE)
    pad2 = padding_vec.reshape(1, EE)
    raws = [
        _gather_fn(W_embed, ids32[g * GR:(g + 1) * GR].reshape(-1))
        .reshape(GR, LL, EE)
        for g in range(NG)
    ]
    out = None
    for g in range(NG):
        out = _pool_project_group(
            raws[g], segment_ids[g * GR:(g + 1) * GR], w_bf, b2, pad2,
            g0=g, prev=out)
    return out


# speedup vs baseline: 1.1243x; 1.1243x over previous
import functools

import jax
import jax.numpy as jnp
from jax import lax
from jax.experimental import pallas as pl
from jax.experimental.pallas import tpu as pltpu
from jax.experimental.pallas import tpu_sc as plsc

BB, LL, EE = 16, 4096, 1024
VV = 32000
NWW = 2048
BM = 512


NG = 4
GR = BB // NG


def _make_gather():
    info = plsc.get_sparse_core_info()
    nc, ns = info.num_cores, info.num_subcores
    nworkers = nc * ns
    per_tile = GR * LL // nworkers
    tbatch = 56
    nfull = per_tile // tbatch
    tail = per_tile - nfull * tbatch

    mesh = plsc.VectorSubcoreMesh(core_axis_name="c", subcore_axis_name="s")

    @functools.partial(
        pl.kernel,
        mesh=mesh,
        out_type=jax.ShapeDtypeStruct((GR * LL, EE), jnp.float32),
        scratch_types=[
            pltpu.VMEM((tbatch,), jnp.int32),
            pltpu.VMEM((tbatch,), jnp.int32),
            pltpu.VMEM((tbatch, EE), jnp.float32),
            pltpu.VMEM((tbatch, EE), jnp.float32),
            pltpu.SemaphoreType.DMA,
            pltpu.SemaphoreType.DMA,
            pltpu.SemaphoreType.DMA,
            pltpu.SemaphoreType.DMA,
        ],
    )
    def gather_k(table, idsflat, out, idx0, idx1, rows0, rows1,
                 gsem0, gsem1, wsem0, wsem1):
        c = lax.axis_index("c")
        s = lax.axis_index("s")
        base = (s * nc + c) * per_tile
        idx = (idx0, idx1)
        rows = (rows0, rows1)
        gsem = (gsem0, gsem1)
        wsem = (wsem0, wsem1)

        def pair_body(g, carry):
            for u in (0, 1):
                i = g * 2 + u

                @pl.when(jnp.logical_and(i >= 2, i < nfull))
                def _():
                    pltpu.make_async_copy(
                        rows[u],
                        out.at[pl.ds(base + (i - 2) * tbatch, tbatch)],
                        wsem[u]).wait()

                @pl.when(i < nfull)
                def _():
                    pltpu.sync_copy(
                        idsflat.at[pl.ds(base + i * tbatch, tbatch)], idx[u])
                    pltpu.async_copy(table.at[idx[u]], rows[u], gsem[u])

                @pl.when(jnp.logical_and(i >= 1, i <= nfull))
                def _():
                    v = 1 - u
                    pltpu.make_async_copy(table.at[idx[v]], rows[v],
                                          gsem[v]).wait()
                    pltpu.async_copy(
                        rows[v],
                        out.at[pl.ds(base + (i - 1) * tbatch, tbatch)],
                        wsem[v])
            return carry

        lax.fori_loop(0, (nfull + 2 + 1) // 2, pair_body, 0)
        pltpu.make_async_copy(
            rows0, out.at[pl.ds(base, tbatch)], wsem0).wait()
        pltpu.make_async_copy(
            rows1, out.at[pl.ds(base, tbatch)], wsem1).wait()
        t0 = base + nfull * tbatch
        pltpu.sync_copy(idsflat.at[pl.ds(t0, tail)], idx0.at[pl.ds(0, tail)])
        pltpu.async_copy(table.at[idx0.at[pl.ds(0, tail)]],
                         rows0.at[pl.ds(0, tail)], gsem0).wait()
        pltpu.sync_copy(rows0.at[pl.ds(0, tail)], out.at[pl.ds(t0, tail)])

    return gather_k


CK = 512


def make_tc_body(bm, ll, ee, ck=CK, has_prev=False):
    def tc_body(raw_ref, seg_ref, w_ref, b_ref, pad_ref, *rest):
        if has_prev:
            _prev_ref, out_ref, acc_ref, cnt_ref = rest
        else:
            out_ref, acc_ref, cnt_ref = rest
        i = pl.program_id(1)
        w0 = i * bm
        wids = lax.broadcasted_iota(jnp.int32, (bm, 1), 0) + w0
        acc_ref[...] = jnp.zeros((bm, ee), jnp.float32)
        cnt_ref[...] = jnp.zeros((bm, 128), jnp.float32)
        for k in range(ll // ck):
            lo_k = seg_ref[0, 0, k * ck]
            hi_k = seg_ref[0, 0, (k + 1) * ck - 1]

            @pl.when(jnp.logical_and(hi_k >= w0, lo_k < w0 + bm))
            def _():
                segc = seg_ref[0, :, k * ck:(k + 1) * ck]
                eqk = segc == wids
                ohk = eqk.astype(jnp.bfloat16)
                rawk = raw_ref[0, k * ck:(k + 1) * ck, :].astype(jnp.bfloat16)
                acc_ref[...] += jnp.dot(ohk, rawk,
                                        preferred_element_type=jnp.float32)
                cntk = jnp.sum(eqk.astype(jnp.float32), axis=1, keepdims=True)
                cnt_ref[...] += jnp.broadcast_to(cntk, (bm, 128))

        cnt = cnt_ref[:, 0:1]
        inv = 1.0 / jnp.maximum(cnt, 1.0)
        y = jnp.dot((acc_ref[...] * inv).astype(jnp.bfloat16), w_ref[...],
                    preferred_element_type=jnp.float32) + b_ref[...]
        out_ref[0] = jnp.where(cnt > 0.0, y, pad_ref[...])
    return tc_body


def _pool_project_group(raw_g, seg_g, w_bf, b2, pad2, g0, prev):
    gr, ll, ee = raw_g.shape
    nw = NWW
    in_specs = [
        pl.BlockSpec((1, ll, ee), lambda b, i: (b, 0, 0)),
        pl.BlockSpec((1, 1, ll), lambda b, i: (b, 0, 0)),
        pl.BlockSpec((ee, ee), lambda b, i: (0, 0)),
        pl.BlockSpec((1, ee), lambda b, i: (0, 0)),
        pl.BlockSpec((1, ee), lambda b, i: (0, 0)),
    ]
    args = [raw_g, seg_g.reshape(gr, 1, ll), w_bf, b2, pad2]
    aliases = {}
    if prev is not None:
        in_specs.append(pl.BlockSpec(memory_space=pl.ANY))
        args.append(prev)
        aliases = {5: 0}
    return pl.pallas_call(
        make_tc_body(BM, ll, ee, has_prev=prev is not None),
        grid=(gr, nw // BM),
        in_specs=in_specs,
        out_specs=pl.BlockSpec((1, BM, ee),
                               lambda b, i: (b + g0 * GR, i, 0)),
        out_shape=jax.ShapeDtypeStruct((BB, nw, ee), jnp.float32),
        input_output_aliases=aliases,
        scratch_shapes=[
            pltpu.VMEM((BM, ee), jnp.float32),
            pltpu.VMEM((BM, 128), jnp.float32),
        ],
    )(*args)


_gather_fn = None


def kernel(subword_ids, segment_ids, W_embed, W_proj, b_proj, padding_vec):
    global _gather_fn
    if _gather_fn is None:
        _gather_fn = _make_gather()
    ids32 = subword_ids.astype(jnp.int32)
    w_bf = W_proj.astype(jnp.bfloat16)
    b2 = b_proj.reshape(1, EE)
    pad2 = padding_vec.reshape(1, EE)
    raws = [
        _gather_fn(W_embed, ids32[g * GR:(g + 1) * GR].reshape(-1))
        .reshape(GR, LL, EE)
        for g in range(NG)
    ]
    out = None
    for g in range(NG):
        out = _pool_project_group(
            raws[g], segment_ids[g * GR:(g + 1) * GR], w_bf, b2, pad2,
            g0=g, prev=out)
    return out
